# baseline (speedup 1.0000x reference)
import jax
import jax.numpy as jnp
from jax import lax
from jax.experimental import pallas as pl
from jax.experimental.pallas import tpu as pltpu

B = 16384
C = 1000
R = 1024
STEPS = B // R

_INT_MIN = -2147483648


def _monotone_key(x):
    b = lax.bitcast_convert_type(x, jnp.int32)
    return jnp.where(b < 0, b ^ jnp.int32(0x7FFFFFFF), b)


def _dual_bottomk_masks(keys, k):
    ukb = keys ^ _INT_MIN
    idx = lax.broadcasted_iota(jnp.int32, (1, B), 1)

    def step(t, carry):
        p, kk = carry
        b = 31 - t
        bitv = lax.shift_left(jnp.int32(1), b)
        above = ~(lax.shift_left(bitv, 1) - 1)
        cand = (ukb & above) == (p & above)
        is0 = (ukb & bitv) == 0
        c0 = jnp.sum((cand & is0).astype(jnp.int32), axis=1, keepdims=True)
        go1 = kk > c0
        p = jnp.where(go1, p | bitv, p)
        kk = jnp.where(go1, kk - c0, kk)
        return p, kk

    p0 = jnp.zeros((2, 1), jnp.int32)
    kk0 = jnp.full((2, 1), k, jnp.int32)
    p, kk = lax.fori_loop(0, 32, step, (p0, kk0))
    tie = ukb == p

    def step2(t, carry):
        p2, kk2 = carry
        b = 13 - t
        bitv = lax.shift_left(jnp.int32(1), b)
        above = ~(lax.shift_left(bitv, 1) - 1)
        cand = tie & ((idx & above) == (p2 & above))
        is0 = (idx & bitv) == 0
        c0 = jnp.sum((cand & is0).astype(jnp.int32), axis=1, keepdims=True)
        go1 = kk2 > c0
        p2 = jnp.where(go1, p2 | bitv, p2)
        kk2 = jnp.where(go1, kk2 - c0, kk2)
        return p2, kk2

    p2, _ = lax.fori_loop(0, 14, step2, (jnp.zeros((2, 1), jnp.int32), kk))
    kT = p ^ _INT_MIN
    return (keys < kT) | ((keys == kT) & (idx <= p2))


def _body(tgt_ref, k_ref, p1_ref, p2_ref, out_ref, l1_ref, l2_ref):
    i = pl.program_id(0)
    tgt = tgt_ref[0, 0, :]
    tgtc = jnp.clip(tgt, 0, C - 1)
    cols = lax.broadcasted_iota(jnp.int32, (R, C), 1)
    eq = cols == tgtc[:, None]

    def ce(x):
        m = jnp.max(x, axis=-1)
        s = jnp.sum(jnp.exp(x - m[:, None]), axis=-1)
        lse = m + jnp.log(s)
        picked = jnp.sum(jnp.where(eq, x, jnp.float32(0.0)), axis=-1)
        return jnp.where(tgt == -1, jnp.float32(0.0), lse - picked)

    l1_ref[pl.ds(i * R, R)] = ce(p1_ref[...])
    l2_ref[pl.ds(i * R, R)] = ce(p2_ref[...])

    @pl.when(i == STEPS - 1)
    def _():
        k = k_ref[0]
        loss1 = l1_ref[...]
        loss2 = l2_ref[...]
        keys = jnp.stack([_monotone_key(loss2), _monotone_key(loss1)])
        keep = _dual_bottomk_masks(keys, k)
        denom = k.astype(jnp.float32)
        out_ref[0] = jnp.sum(jnp.where(keep[0, :], loss1, jnp.float32(0.0))) / denom
        out_ref[1] = jnp.sum(jnp.where(keep[1, :], loss2, jnp.float32(0.0))) / denom


def kernel(preds1, preds2, target, forget_rate):
    n = preds1.shape[0]
    num_remember = jnp.int32(n) - jnp.ceil(forget_rate * n).astype(jnp.int32)
    k_arr = num_remember.reshape(1)
    target3 = target.reshape(STEPS, 1, R)
    out = pl.pallas_call(
        _body,
        grid=(STEPS,),
        in_specs=[
            pl.BlockSpec((1, 1, R), lambda i: (i, 0, 0)),
            pl.BlockSpec(memory_space=pltpu.SMEM),
            pl.BlockSpec((R, C), lambda i: (i, 0)),
            pl.BlockSpec((R, C), lambda i: (i, 0)),
        ],
        out_specs=pl.BlockSpec(memory_space=pltpu.SMEM),
        out_shape=jax.ShapeDtypeStruct((2,), jnp.float32),
        scratch_shapes=[
            pltpu.VMEM((B,), jnp.float32),
            pltpu.VMEM((B,), jnp.float32),
        ],
        compiler_params=pltpu.CompilerParams(
            dimension_semantics=("arbitrary",)),
    )(target3, k_arr, preds1, preds2)
    return (out[0], out[1])
